# initial kernel scaffold (unmeasured)
import jax
import jax.numpy as jnp
from jax import lax
from jax.experimental import pallas as pl
from jax.experimental.pallas import tpu as pltpu

N_DEV = 16
M = 4096
N_OUT = 2048
CHUNK = M // N_DEV


def kernel(x, w_mat):
    m, k_per = x.shape
    _, n = w_mat.shape

    def body(x_ref, w_ref, out_ref, recv_buf, send_sems, recv_sems, credit_sem):
        my = lax.axis_index("i")
        left = (my - 1) % N_DEV
        right = (my + 1) % N_DEV

        barrier_sem = pltpu.get_barrier_semaphore()
        for nbr in (left, right):
            pl.semaphore_signal(
                barrier_sem, inc=1,
                device_id=(nbr,), device_id_type=pl.DeviceIdType.MESH,
            )
        pl.semaphore_wait(barrier_sem, 2)

        out_ref[:, :] = jnp.dot(
            x_ref[:, :], w_ref[:, :], preferred_element_type=jnp.float32
        )

        def rows(c):
            return pl.ds(c * CHUNK, CHUNK)

        for s in range(N_DEV - 1):
            slot = s % 2
            send_c = (my - s) % N_DEV
            recv_c = (my - s - 1) % N_DEV
            if s >= 2:
                pl.semaphore_wait(credit_sem, 1)
            rdma = pltpu.make_async_remote_copy(
                src_ref=out_ref.at[rows(send_c), :],
                dst_ref=recv_buf.at[slot],
                send_sem=send_sems.at[slot],
                recv_sem=recv_sems.at[slot],
                device_id=(right,),
                device_id_type=pl.DeviceIdType.MESH,
            )
            rdma.start()
            rdma.wait()
            out_ref[rows(recv_c), :] = out_ref[rows(recv_c), :] + recv_buf[slot]
            pl.semaphore_signal(
                credit_sem, inc=1,
                device_id=(left,), device_id_type=pl.DeviceIdType.MESH,
            )


        for t in range(N_DEV - 1):
            s = (N_DEV - 1) + t
            slot = s % 2
            send_c = (my + 1 - t) % N_DEV
            recv_c = (my - t) % N_DEV
            pl.semaphore_wait(credit_sem, 1)
            rdma = pltpu.make_async_remote_copy(
                src_ref=out_ref.at[rows(send_c), :],
                dst_ref=out_ref.at[rows(send_c), :],
                send_sem=send_sems.at[slot],
                recv_sem=recv_sems.at[slot],
                device_id=(right,),
                device_id_type=pl.DeviceIdType.MESH,
            )
            rdma.start()
            rdma.wait()
            if t < N_DEV - 3:
                pl.semaphore_signal(
                    credit_sem, inc=1,
                    device_id=(left,), device_id_type=pl.DeviceIdType.MESH,
                )

    return pl.pallas_call(
        body,
        out_shape=jax.ShapeDtypeStruct((M, N_OUT), jnp.float32),
        in_specs=[
            pl.BlockSpec(memory_space=pltpu.VMEM),
            pl.BlockSpec(memory_space=pltpu.VMEM),
        ],
        out_specs=pl.BlockSpec(memory_space=pltpu.VMEM),
        scratch_shapes=[
            pltpu.VMEM((2, CHUNK, N_OUT), jnp.float32),
            pltpu.SemaphoreType.DMA((2,)),
            pltpu.SemaphoreType.DMA((2,)),
            pltpu.SemaphoreType.REGULAR,
        ],
        compiler_params=pltpu.CompilerParams(collective_id=0),
    )(x, w_mat)


# baseline (device time: 777663 ns/iter reference)
import jax
import jax.numpy as jnp
from jax import lax
from jax.experimental import pallas as pl
from jax.experimental.pallas import tpu as pltpu

N_DEV = 16
M = 4096
N_OUT = 2048
CHUNK = M // N_DEV


def kernel(x, w_mat):
    m, k_per = x.shape
    _, n = w_mat.shape

    def body(x_ref, w_ref, out_ref, recv_buf, send_sems, recv_sems, credit_sem):
        my = lax.axis_index("i")
        left = (my - 1) % N_DEV
        right = (my + 1) % N_DEV

        barrier_sem = pltpu.get_barrier_semaphore()
        for nbr in (left, right):
            pl.semaphore_signal(
                barrier_sem, inc=1,
                device_id=(nbr,), device_id_type=pl.DeviceIdType.MESH,
            )
        pl.semaphore_wait(barrier_sem, 2)

        out_ref[:, :] = jnp.dot(
            x_ref[:, :], w_ref[:, :], preferred_element_type=jnp.float32
        )

        def rows(c):
            return pl.ds(c * CHUNK, CHUNK)

        for s in range(N_DEV - 1):
            slot = s % 2
            send_c = (my - s) % N_DEV
            recv_c = (my - s - 1) % N_DEV
            if s >= 2:
                pl.semaphore_wait(credit_sem, 1)
            rdma = pltpu.make_async_remote_copy(
                src_ref=out_ref.at[rows(send_c), :],
                dst_ref=recv_buf.at[slot],
                send_sem=send_sems.at[slot],
                recv_sem=recv_sems.at[slot],
                device_id=(right,),
                device_id_type=pl.DeviceIdType.MESH,
            )
            rdma.start()
            rdma.wait()
            out_ref[rows(recv_c), :] = out_ref[rows(recv_c), :] + recv_buf[slot]
            pl.semaphore_signal(
                credit_sem, inc=1,
                device_id=(left,), device_id_type=pl.DeviceIdType.MESH,
            )


        for t in range(N_DEV - 1):
            s = (N_DEV - 1) + t
            slot = s % 2
            send_c = (my + 1 - t) % N_DEV
            recv_c = (my - t) % N_DEV
            pl.semaphore_wait(credit_sem, 1)
            rdma = pltpu.make_async_remote_copy(
                src_ref=out_ref.at[rows(send_c), :],
                dst_ref=out_ref.at[rows(send_c), :],
                send_sem=send_sems.at[slot],
                recv_sem=recv_sems.at[slot],
                device_id=(right,),
                device_id_type=pl.DeviceIdType.MESH,
            )
            rdma.start()
            rdma.wait()
            if t < N_DEV - 3:
                pl.semaphore_signal(
                    credit_sem, inc=1,
                    device_id=(left,), device_id_type=pl.DeviceIdType.MESH,
                )

    return pl.pallas_call(
        body,
        out_shape=jax.ShapeDtypeStruct((M, N_OUT), jnp.float32),
        in_specs=[
            pl.BlockSpec(memory_space=pltpu.VMEM),
            pl.BlockSpec(memory_space=pltpu.VMEM),
        ],
        out_specs=pl.BlockSpec(memory_space=pltpu.VMEM),
        scratch_shapes=[
            pltpu.VMEM((2, CHUNK, N_OUT), jnp.float32),
            pltpu.SemaphoreType.DMA((2,)),
            pltpu.SemaphoreType.DMA((2,)),
            pltpu.SemaphoreType.REGULAR,
        ],
        compiler_params=pltpu.CompilerParams(
            collective_id=0, vmem_limit_bytes=100 * 1024 * 1024
        ),
    )(x, w_mat)


# device time: 475566 ns/iter; 1.6352x vs baseline; 1.6352x over previous
import jax
import jax.numpy as jnp
from jax import lax
from jax.experimental import pallas as pl
from jax.experimental.pallas import tpu as pltpu

N_DEV = 16
M = 4096
N_OUT = 2048
CHUNK = M // N_DEV
HALF = N_OUT // 2


def kernel(x, w_mat):
    def body(
        x_ref, w_ref, out_ref,
        rbuf_cw, rbuf_ccw,
        ssem_cw, rsem_cw, ssem_ccw, rsem_ccw,
        cred_cw, cred_ccw,
    ):
        my = lax.axis_index("i")
        left = (my - 1) % N_DEV
        right = (my + 1) % N_DEV

        barrier_sem = pltpu.get_barrier_semaphore()
        for nbr in (left, right):
            pl.semaphore_signal(
                barrier_sem, inc=1,
                device_id=(nbr,), device_id_type=pl.DeviceIdType.MESH,
            )
        pl.semaphore_wait(barrier_sem, 2)

        def rows(c):
            return pl.ds(c * CHUNK, CHUNK)

        def compute_chunk(c):
            out_ref[rows(c), :] = jnp.dot(
                x_ref[rows(c), :], w_ref[:, :],
                preferred_element_type=jnp.float32,
            )

        compute_chunk(my)

        n_steps = 2 * (N_DEV - 1)
        for s in range(n_steps):
            slot = s % 2
            is_rs = s < N_DEV - 1
            t = s - (N_DEV - 1)
            if is_rs:
                cw_send = (my - s) % N_DEV
                cw_recv = (my - s - 1) % N_DEV
                ccw_send = (my + s) % N_DEV
                ccw_recv = (my + s + 1) % N_DEV
            else:
                cw_send = (my + 1 - t) % N_DEV
                cw_recv = (my - t) % N_DEV
                ccw_send = (my - 1 + t) % N_DEV
                ccw_recv = (my + t) % N_DEV

            if s >= 2:
                pl.semaphore_wait(cred_cw, 1)
                pl.semaphore_wait(cred_ccw, 1)

            cw = pltpu.make_async_remote_copy(
                src_ref=out_ref.at[rows(cw_send), pl.ds(0, HALF)],
                dst_ref=(
                    rbuf_cw.at[slot] if is_rs
                    else out_ref.at[rows(cw_send), pl.ds(0, HALF)]
                ),
                send_sem=ssem_cw.at[slot],
                recv_sem=rsem_cw.at[slot],
                device_id=(right,),
                device_id_type=pl.DeviceIdType.MESH,
            )
            ccw = pltpu.make_async_remote_copy(
                src_ref=out_ref.at[rows(ccw_send), pl.ds(HALF, HALF)],
                dst_ref=(
                    rbuf_ccw.at[slot] if is_rs
                    else out_ref.at[rows(ccw_send), pl.ds(HALF, HALF)]
                ),
                send_sem=ssem_ccw.at[slot],
                recv_sem=rsem_ccw.at[slot],
                device_id=(left,),
                device_id_type=pl.DeviceIdType.MESH,
            )
            cw.start()
            ccw.start()

            if is_rs and s <= 6:
                compute_chunk((my - s - 1) % N_DEV)
                compute_chunk((my + s + 1) % N_DEV)
            elif is_rs and s == 7:
                compute_chunk((my + 8) % N_DEV)

            cw.wait()
            if is_rs:
                out_ref[rows(cw_recv), :HALF] = (
                    out_ref[rows(cw_recv), :HALF] + rbuf_cw[slot]
                )
            ccw.wait()
            if is_rs:
                out_ref[rows(ccw_recv), HALF:] = (
                    out_ref[rows(ccw_recv), HALF:] + rbuf_ccw[slot]
                )

            if s < n_steps - 2:
                pl.semaphore_signal(
                    cred_cw, inc=1,
                    device_id=(left,), device_id_type=pl.DeviceIdType.MESH,
                )
                pl.semaphore_signal(
                    cred_ccw, inc=1,
                    device_id=(right,), device_id_type=pl.DeviceIdType.MESH,
                )

    return pl.pallas_call(
        body,
        out_shape=jax.ShapeDtypeStruct((M, N_OUT), jnp.float32),
        in_specs=[
            pl.BlockSpec(memory_space=pltpu.VMEM),
            pl.BlockSpec(memory_space=pltpu.VMEM),
        ],
        out_specs=pl.BlockSpec(memory_space=pltpu.VMEM),
        scratch_shapes=[
            pltpu.VMEM((2, CHUNK, HALF), jnp.float32),
            pltpu.VMEM((2, CHUNK, HALF), jnp.float32),
            pltpu.SemaphoreType.DMA((2,)),
            pltpu.SemaphoreType.DMA((2,)),
            pltpu.SemaphoreType.DMA((2,)),
            pltpu.SemaphoreType.DMA((2,)),
            pltpu.SemaphoreType.REGULAR,
            pltpu.SemaphoreType.REGULAR,
        ],
        compiler_params=pltpu.CompilerParams(
            collective_id=0, vmem_limit_bytes=100 * 1024 * 1024
        ),
    )(x, w_mat)


# device time: 383245 ns/iter; 2.0292x vs baseline; 1.2409x over previous
import jax
import jax.numpy as jnp
from jax import lax
from jax.experimental import pallas as pl
from jax.experimental.pallas import tpu as pltpu

N_DEV = 16
M = 4096
N_OUT = 2048
CHUNK = M // N_DEV
SEG = N_OUT // 4
N_STEPS = 2 * (N_DEV - 1)


def kernel(x, w_mat):
    def body(
        x_ref, w_ref, out_ref,
        rbuf0, rbuf1, rbuf2, rbuf3,
        ssem0, ssem1, ssem2, ssem3,
        rsem0, rsem1, rsem2, rsem3,
        cred0, cred1, cred2, cred3,
    ):
        my = lax.axis_index("i")
        left = (my - 1) % N_DEV
        right = (my + 1) % N_DEV

        rings = [
            (0 * SEG, True, rbuf0, ssem0, rsem0, cred0),
            (2 * SEG, False, rbuf2, ssem2, rsem2, cred2),
            (1 * SEG, True, rbuf1, ssem1, rsem1, cred1),
            (3 * SEG, False, rbuf3, ssem3, rsem3, cred3),
        ]

        barrier_sem = pltpu.get_barrier_semaphore()
        for nbr in (left, right):
            pl.semaphore_signal(
                barrier_sem, inc=1,
                device_id=(nbr,), device_id_type=pl.DeviceIdType.MESH,
            )
        pl.semaphore_wait(barrier_sem, 2)

        def rows(c):
            return pl.ds(c * CHUNK, CHUNK)

        def compute_chunk(c):
            out_ref[rows(c), :] = jnp.dot(
                x_ref[rows(c), :], w_ref[:, :],
                preferred_element_type=jnp.float32,
            )

        def chunk_ids(u, is_cw):
            if u < N_DEV - 1:
                if is_cw:
                    return (my - u) % N_DEV, (my - u - 1) % N_DEV
                return (my + u) % N_DEV, (my + u + 1) % N_DEV
            t = u - (N_DEV - 1)
            if is_cw:
                return (my + 1 - t) % N_DEV, (my - t) % N_DEV
            return (my - 1 + t) % N_DEV, (my + t) % N_DEV

        def make_desc(u, off, is_cw, rbuf, ssem, rsem):
            slot = u % 2
            send_c, recv_c = chunk_ids(u, is_cw)
            if u < N_DEV - 1:
                dst = rbuf.at[slot]
            else:
                dst = out_ref.at[rows(send_c), pl.ds(off, SEG)]
            return pltpu.make_async_remote_copy(
                src_ref=out_ref.at[rows(send_c), pl.ds(off, SEG)],
                dst_ref=dst,
                send_sem=ssem.at[slot],
                recv_sem=rsem.at[slot],
                device_id=(right if is_cw else left,),
                device_id_type=pl.DeviceIdType.MESH,
            )

        compute_chunk(my)

        descs = {off: {} for off, *_ in rings}
        for off, is_cw, rbuf, ssem, rsem, _cred in rings:
            d = make_desc(0, off, is_cw, rbuf, ssem, rsem)
            descs[off][0] = d
            d.start()

        for s in range(N_STEPS):
            if s <= 6:
                compute_chunk((my - s - 1) % N_DEV)
                compute_chunk((my + s + 1) % N_DEV)
            elif s == 7:
                compute_chunk((my + 8) % N_DEV)

            for off, is_cw, rbuf, ssem, rsem, cred in rings:
                slot = s % 2
                _send_c, recv_c = chunk_ids(s, is_cw)
                descs[off][s].wait_recv()
                if s < N_DEV - 1:
                    out_ref[rows(recv_c), pl.ds(off, SEG)] = (
                        out_ref[rows(recv_c), pl.ds(off, SEG)] + rbuf[slot]
                    )
                if s < N_STEPS - 2:
                    pl.semaphore_signal(
                        cred, inc=1,
                        device_id=(left if is_cw else right,),
                        device_id_type=pl.DeviceIdType.MESH,
                    )
                if s + 1 < N_STEPS:
                    u = s + 1
                    if u >= 2:
                        pl.semaphore_wait(cred, 1)
                        descs[off][u - 2].wait_send()
                    d = make_desc(u, off, is_cw, rbuf, ssem, rsem)
                    descs[off][u] = d
                    d.start()

        for off, *_ in rings:
            descs[off][N_STEPS - 2].wait_send()
            descs[off][N_STEPS - 1].wait_send()

    return pl.pallas_call(
        body,
        out_shape=jax.ShapeDtypeStruct((M, N_OUT), jnp.float32),
        in_specs=[
            pl.BlockSpec(memory_space=pltpu.VMEM),
            pl.BlockSpec(memory_space=pltpu.VMEM),
        ],
        out_specs=pl.BlockSpec(memory_space=pltpu.VMEM),
        scratch_shapes=[
            pltpu.VMEM((2, CHUNK, SEG), jnp.float32),
            pltpu.VMEM((2, CHUNK, SEG), jnp.float32),
            pltpu.VMEM((2, CHUNK, SEG), jnp.float32),
            pltpu.VMEM((2, CHUNK, SEG), jnp.float32),
            pltpu.SemaphoreType.DMA((2,)),
            pltpu.SemaphoreType.DMA((2,)),
            pltpu.SemaphoreType.DMA((2,)),
            pltpu.SemaphoreType.DMA((2,)),
            pltpu.SemaphoreType.DMA((2,)),
            pltpu.SemaphoreType.DMA((2,)),
            pltpu.SemaphoreType.DMA((2,)),
            pltpu.SemaphoreType.DMA((2,)),
            pltpu.SemaphoreType.REGULAR,
            pltpu.SemaphoreType.REGULAR,
            pltpu.SemaphoreType.REGULAR,
            pltpu.SemaphoreType.REGULAR,
        ],
        compiler_params=pltpu.CompilerParams(
            collective_id=0, vmem_limit_bytes=100 * 1024 * 1024
        ),
    )(x, w_mat)
